# baseline (speedup 1.0000x reference)
import math
import functools

import jax
import jax.numpy as jnp
from jax import lax
from jax.experimental import pallas as pl
from jax.experimental.pallas import tpu as pltpu
from jax.experimental.pallas import tpu_sc as plsc

NC = 2
NS = 16
NW = NC * NS
LANES = 16
NG = 4


def _make_kernel(B, S, D, CH):
    SPW = S // NW
    T = (B * SPW) // CH
    scale = 1.0 / math.sqrt(D)
    VPR = D // LANES
    assert T % NG == 0 and SPW % CH == 0
    mesh = plsc.VectorSubcoreMesh(
        core_axis_name="c", subcore_axis_name="s",
        num_cores=NC, num_subcores=NS)

    @functools.partial(
        pl.kernel,
        out_type=jax.ShapeDtypeStruct((B * S, D), jnp.float32),
        mesh=mesh,
        scratch_types=[
            pltpu.VMEM((B * SPW,), jnp.int32),
            pltpu.VMEM((CH, D), jnp.float32),
            pltpu.VMEM((CH, D), jnp.float32),
            pltpu.VMEM((CH, D), jnp.float32),
            pltpu.VMEM((CH, D), jnp.float32),
            pltpu.VMEM((CH, D), jnp.float32),
            pltpu.VMEM((CH, D), jnp.float32),
            pltpu.VMEM((CH, D), jnp.float32),
            pltpu.SemaphoreType.DMA,
            pltpu.SemaphoreType.DMA,
            pltpu.SemaphoreType.DMA,
            pltpu.SemaphoreType.DMA,
            pltpu.SemaphoreType.DMA,
            pltpu.SemaphoreType.DMA,
        ],
    )
    def k(x_hbm, tok_hbm, pos_hbm, out_hbm,
          idx_v, pos_v, g0, g1, g2, g3, oa, ob,
          gs0, gs1, gs2, gs3, ws0, ws1):
        wid = lax.axis_index("s") * NC + lax.axis_index("c")
        s_base = wid * SPW
        gbuf = (g0, g1, g2, g3)
        gsem = (gs0, gs1, gs2, gs3)
        obuf = (oa, ob)
        wsem = (ws0, ws1)

        for b in range(B):
            pltpu.sync_copy(x_hbm.at[pl.ds(b * S + s_base, SPW)],
                            idx_v.at[pl.ds(b * SPW, SPW)])
        pltpu.sync_copy(pos_hbm.at[pl.ds(s_base, CH), :], pos_v)

        def start_gather(t, ring):
            b = t % B
            s_off = (t // B) * CH
            pltpu.async_copy(
                tok_hbm.at[idx_v.at[pl.ds(b * SPW + s_off, CH)]],
                gbuf[ring], gsem[ring])

        for i in range(NG):
            start_gather(i, i)

        @pl.loop(0, T, step=NG)
        def _grp(t0):
            for kk in range(NG):
                t = t0 + kk
                g = gbuf[kk]
                o = obuf[kk % 2]

                if kk == 0:
                    @pl.when(t > 0)
                    def _():
                        pltpu.sync_copy(
                            pos_hbm.at[pl.ds(s_base + (t // B) * CH, CH), :],
                            pos_v)

                pltpu.make_async_copy(
                    tok_hbm.at[idx_v.at[pl.ds(0, CH)]], g, gsem[kk]).wait()

                @pl.when(t >= 2)
                def _():
                    pltpu.make_async_copy(
                        o, out_hbm.at[pl.ds(0, CH), :], wsem[kk % 2]).wait()

                @plsc.parallel_loop(0, CH * VPR, unroll=8)
                def _v(i):
                    r = i // VPR
                    cs = (i % VPR) * LANES
                    o[r, pl.ds(cs, LANES)] = (
                        g[r, pl.ds(cs, LANES)] * scale
                        + pos_v[r, pl.ds(cs, LANES)])

                b = t % B
                s_off = (t // B) * CH
                pltpu.async_copy(
                    o, out_hbm.at[pl.ds(b * S + s_base + s_off, CH), :],
                    wsem[kk % 2])

                @pl.when(t + NG < T)
                def _():
                    start_gather(t + NG, kk)

        for j in range(2):
            pltpu.make_async_copy(obuf[j], out_hbm.at[pl.ds(0, CH), :],
                                  wsem[j]).wait()

    return k


@jax.jit
def kernel(x, tok_emb, pos_emb):
    B, S = x.shape
    D = tok_emb.shape[1]
    xf = x.reshape(B * S).astype(jnp.int32)
    out = _make_kernel(B, S, D, CH=16)(xf, tok_emb, pos_emb)
    return out.reshape(B, S, D)


# speedup vs baseline: 1.0989x; 1.0989x over previous
import math
import functools

import jax
import jax.numpy as jnp
from jax import lax
from jax.experimental import pallas as pl
from jax.experimental.pallas import tpu as pltpu
from jax.experimental.pallas import tpu_sc as plsc

NC = 2
NS = 16
NW = NC * NS
LANES = 16


def _make_kernel(B, S, D, CH):
    SPW = S // NW
    G = SPW // CH
    scale = 1.0 / math.sqrt(D)
    VPR = D // LANES
    assert G % 2 == 0 and B == 4
    mesh = plsc.VectorSubcoreMesh(
        core_axis_name="c", subcore_axis_name="s",
        num_cores=NC, num_subcores=NS)

    @functools.partial(
        pl.kernel,
        out_type=jax.ShapeDtypeStruct((B * S, D), jnp.float32),
        mesh=mesh,
        scratch_types=[
            pltpu.VMEM((B * SPW,), jnp.int32),
            pltpu.VMEM((2, CH, D), jnp.float32),
        ] + [pltpu.VMEM((CH, D), jnp.float32) for _ in range(8)]
          + [pltpu.VMEM((CH, D), jnp.float32) for _ in range(4)]
          + [pltpu.SemaphoreType.DMA for _ in range(8)]
          + [pltpu.SemaphoreType.DMA for _ in range(4)]
          + [pltpu.SemaphoreType.DMA for _ in range(2)],
    )
    def k(x_hbm, tok_hbm, pos_hbm, out_hbm, idx_v, pos_v, *rest):
        gbuf = rest[0:8]
        obuf = rest[8:12]
        gsem = rest[12:20]
        wsem = rest[20:24]
        psem = rest[24:26]
        wid = lax.axis_index("s") * NC + lax.axis_index("c")
        s_base = wid * SPW

        for b in range(B):
            pltpu.sync_copy(x_hbm.at[pl.ds(b * S + s_base, SPW)],
                            idx_v.at[pl.ds(b * SPW, SPW)])

        def start_pos(g, par):
            pltpu.async_copy(
                pos_hbm.at[pl.ds(s_base + g * CH, CH), :],
                pos_v.at[par], psem[par])

        def wait_pos(par):
            pltpu.make_async_copy(
                pos_hbm.at[pl.ds(0, CH), :], pos_v.at[par], psem[par]).wait()

        def start_gathers(g, par):
            for j in range(B):
                pltpu.async_copy(
                    tok_hbm.at[idx_v.at[pl.ds(j * SPW + g * CH, CH)]],
                    gbuf[par * 4 + j], gsem[par * 4 + j])

        start_pos(0, 0)
        start_pos(1, 1)
        start_gathers(0, 0)
        start_gathers(1, 1)

        @pl.loop(0, G, step=2)
        def _pair(g0):
            for par in range(2):
                g = g0 + par

                for j in range(B):
                    pltpu.make_async_copy(
                        tok_hbm.at[idx_v.at[pl.ds(0, CH)]],
                        gbuf[par * 4 + j], gsem[par * 4 + j]).wait()
                wait_pos(par)

                @pl.when(g >= 1)
                def _():
                    for j in range(B):
                        pltpu.make_async_copy(
                            obuf[j], out_hbm.at[pl.ds(0, CH), :],
                            wsem[j]).wait()

                ps = pos_v.at[par]
                gs = [gbuf[par * 4 + j] for j in range(B)]

                @plsc.parallel_loop(0, CH * VPR, unroll=4)
                def _v(i):
                    r = i // VPR
                    cs = (i % VPR) * LANES
                    p = ps[r, pl.ds(cs, LANES)]
                    for j in range(B):
                        obuf[j][r, pl.ds(cs, LANES)] = (
                            gs[j][r, pl.ds(cs, LANES)] * scale + p)

                for j in range(B):
                    pltpu.async_copy(
                        obuf[j],
                        out_hbm.at[pl.ds(j * S + s_base + g * CH, CH), :],
                        wsem[j])

                @pl.when(g + 2 < G)
                def _():
                    start_pos(g + 2, par)
                    start_gathers(g + 2, par)

        for j in range(B):
            pltpu.make_async_copy(obuf[j], out_hbm.at[pl.ds(0, CH), :],
                                  wsem[j]).wait()

    return k


@jax.jit
def kernel(x, tok_emb, pos_emb):
    B, S = x.shape
    D = tok_emb.shape[1]
    xf = x.reshape(B * S).astype(jnp.int32)
    out = _make_kernel(B, S, D, CH=8)(xf, tok_emb, pos_emb)
    return out.reshape(B, S, D)


# speedup vs baseline: 1.1139x; 1.0137x over previous
import math
import functools

import jax
import jax.numpy as jnp
from jax import lax
from jax.experimental import pallas as pl
from jax.experimental.pallas import tpu as pltpu
from jax.experimental.pallas import tpu_sc as plsc

NC = 2
NS = 16
NW = NC * NS
LANES = 16


def _make_kernel(B, S, D, CH):
    SPW = S // NW
    G = SPW // CH
    scale = 1.0 / math.sqrt(D)
    VPR = D // LANES
    assert G % 2 == 0 and B == 4
    mesh = plsc.VectorSubcoreMesh(
        core_axis_name="c", subcore_axis_name="s",
        num_cores=NC, num_subcores=NS)

    @functools.partial(
        pl.kernel,
        out_type=jax.ShapeDtypeStruct((B * S, D), jnp.float32),
        mesh=mesh,
        scratch_types=[
            pltpu.VMEM((B * SPW,), jnp.int32),
            pltpu.VMEM((2, CH, D), jnp.float32),
        ] + [pltpu.VMEM((CH, D), jnp.float32) for _ in range(8)]
          + [pltpu.VMEM((CH, D), jnp.float32) for _ in range(4)]
          + [pltpu.SemaphoreType.DMA for _ in range(8)]
          + [pltpu.SemaphoreType.DMA for _ in range(4)]
          + [pltpu.SemaphoreType.DMA for _ in range(2)],
    )
    def k(x_hbm, tok_hbm, pos_hbm, out_hbm, idx_v, pos_v, *rest):
        gbuf = rest[0:8]
        obuf = rest[8:12]
        gsem = rest[12:20]
        wsem = rest[20:24]
        psem = rest[24:26]
        wid = lax.axis_index("s") * NC + lax.axis_index("c")
        s_base = wid * SPW

        def start_pos(g, par):
            pltpu.async_copy(
                pos_hbm.at[pl.ds(s_base + g * CH, CH), :],
                pos_v.at[par], psem[par])

        def wait_pos(par):
            pltpu.make_async_copy(
                pos_hbm.at[pl.ds(0, CH), :], pos_v.at[par], psem[par]).wait()

        def start_gathers(g, par):
            for j in range(B):
                pltpu.async_copy(
                    tok_hbm.at[idx_v.at[pl.ds(j * SPW + g * CH, CH)]],
                    gbuf[par * 4 + j], gsem[par * 4 + j])

        start_pos(0, 0)
        start_pos(1, 1)
        for b in range(B):
            pltpu.sync_copy(x_hbm.at[pl.ds(b * S + s_base, SPW)],
                            idx_v.at[pl.ds(b * SPW, SPW)])
        start_gathers(0, 0)
        start_gathers(1, 1)

        @pl.loop(0, G, step=2)
        def _pair(g0):
            for par in range(2):
                g = g0 + par

                for j in range(B):
                    pltpu.make_async_copy(
                        tok_hbm.at[idx_v.at[pl.ds(0, CH)]],
                        gbuf[par * 4 + j], gsem[par * 4 + j]).wait()
                wait_pos(par)

                @pl.when(g >= 1)
                def _():
                    for j in range(B):
                        pltpu.make_async_copy(
                            obuf[j], out_hbm.at[pl.ds(0, CH), :],
                            wsem[j]).wait()

                ps = pos_v.at[par]
                gs = [gbuf[par * 4 + j] for j in range(B)]

                @plsc.parallel_loop(0, CH * VPR, unroll=8)
                def _v(i):
                    r = i // VPR
                    cs = (i % VPR) * LANES
                    p = ps[r, pl.ds(cs, LANES)]
                    for j in range(B):
                        obuf[j][r, pl.ds(cs, LANES)] = (
                            gs[j][r, pl.ds(cs, LANES)] * scale + p)

                for j in range(B):
                    pltpu.async_copy(
                        obuf[j],
                        out_hbm.at[pl.ds(j * S + s_base + g * CH, CH), :],
                        wsem[j])

                @pl.when(g + 2 < G)
                def _():
                    start_pos(g + 2, par)
                    start_gathers(g + 2, par)

        for j in range(B):
            pltpu.make_async_copy(obuf[j], out_hbm.at[pl.ds(0, CH), :],
                                  wsem[j]).wait()

    return k


@jax.jit
def kernel(x, tok_emb, pos_emb):
    B, S = x.shape
    D = tok_emb.shape[1]
    xf = x.reshape(B * S).astype(jnp.int32)
    out = _make_kernel(B, S, D, CH=8)(xf, tok_emb, pos_emb)
    return out.reshape(B, S, D)
